# baseline (speedup 1.0000x reference)
import jax
import jax.numpy as jnp
from jax.experimental import pallas as pl
from jax.experimental.pallas import tpu as pltpu

NUM_EXPERTS = 8
TOP_K = 2
HIDDEN = 768
INTER = 2048
T = 2048

TR = 128
NT = (2 * T + NUM_EXPERTS * TR) // TR
P = NT * TR


def _cumsum_manual(a, axis, n):
    idx = jax.lax.broadcasted_iota(jnp.int32, a.shape, axis)
    k = 1
    while k < n:
        a = a + jnp.where(idx >= k, jnp.roll(a, k, axis=axis), 0)
        k *= 2
    return a


def _route_kernel(x_ref, rw_ref, p_ref, w_ref, eot_ref):
    logits = jax.lax.dot_general(
        rw_ref[...], x_ref[...], (((1,), (1,)), ((), ())),
        preferred_element_type=jnp.float32)
    m = jnp.max(logits, axis=0, keepdims=True)
    ex = jnp.exp(logits - m)
    probs = ex / jnp.sum(ex, axis=0, keepdims=True)

    eidx = jax.lax.broadcasted_iota(jnp.int32, (NUM_EXPERTS, T), 0)
    w0 = jnp.max(probs, axis=0, keepdims=True)
    e0 = jnp.min(jnp.where(probs == w0, eidx, NUM_EXPERTS), axis=0,
                 keepdims=True)
    oh0 = (eidx == e0).astype(jnp.int32)
    probs2 = jnp.where(oh0 == 1, -1.0, probs)
    w1 = jnp.max(probs2, axis=0, keepdims=True)
    e1 = jnp.min(jnp.where(probs2 == w1, eidx, NUM_EXPERTS), axis=0,
                 keepdims=True)
    oh1 = (eidx == e1).astype(jnp.int32)

    cum0 = _cumsum_manual(oh0, 1, T)
    cum1 = _cumsum_manual(oh1, 1, T)
    counts0 = jnp.sum(oh0, axis=1, keepdims=True)
    counts = counts0 + jnp.sum(oh1, axis=1, keepdims=True)
    cnt_pad = ((counts + TR - 1) // TR) * TR
    pad_off = _cumsum_manual(cnt_pad, 0, NUM_EXPERTS) - cnt_pad

    rank0 = jnp.sum(oh0 * (cum0 - 1), axis=0, keepdims=True)
    rank1 = jnp.sum(oh1 * (cum1 - 1 + counts0), axis=0, keepdims=True)
    base0 = jnp.sum(oh0 * pad_off, axis=0, keepdims=True)
    base1 = jnp.sum(oh1 * pad_off, axis=0, keepdims=True)

    p_ref[0:1, :] = base0 + rank0
    p_ref[1:2, :] = base1 + rank1
    w_ref[0:1, :] = w0
    w_ref[1:2, :] = w1

    tstart = jax.lax.broadcasted_iota(jnp.int32, (1, NT), 1) * TR
    eot = jnp.sum((tstart >= pad_off).astype(jnp.int32), axis=0,
                  keepdims=True) - 1
    eot_ref[...] = jnp.clip(eot, 0, NUM_EXPERTS - 1)


def _gmm_kernel(eot_ref, p_ref, w_ref, xb_ref, wg_ref, wu_ref, w2_ref,
                out_ref):
    j = pl.program_id(0)

    @pl.when(j == 0)
    def _init():
        out_ref[...] = jnp.zeros_like(out_ref)

    rr = jax.lax.broadcasted_iota(jnp.int32, (TR, T), 0) + j * TR
    m0 = p_ref[0:1, :] == rr
    m1 = p_ref[1:2, :] == rr
    ohu = (m0 | m1).astype(jnp.bfloat16)
    ohw = (jnp.where(m0, w_ref[0:1, :], 0.0)
           + jnp.where(m1, w_ref[1:2, :], 0.0)).astype(jnp.bfloat16)

    xs = jax.lax.dot_general(
        ohu, xb_ref[...], (((1,), (0,)), ((), ())),
        preferred_element_type=jnp.float32).astype(jnp.bfloat16)
    g = jax.lax.dot_general(
        xs, wg_ref[0], (((1,), (1,)), ((), ())),
        preferred_element_type=jnp.float32)
    u = jax.lax.dot_general(
        xs, wu_ref[0], (((1,), (1,)), ((), ())),
        preferred_element_type=jnp.float32)
    h = (g * jax.lax.logistic(g) * u).astype(jnp.bfloat16)
    y = jax.lax.dot_general(
        h, w2_ref[0], (((1,), (1,)), ((), ())),
        preferred_element_type=jnp.float32).astype(jnp.bfloat16)
    out_ref[...] += jax.lax.dot_general(
        ohw, y, (((0,), (0,)), ((), ())),
        preferred_element_type=jnp.float32)


@jax.jit
def kernel(hidden_states, router_w, w13, w2):
    orig_shape = hidden_states.shape
    x = hidden_states.reshape(T, HIDDEN)
    xb = x.astype(jnp.bfloat16)
    wg = w13[:, :INTER, :].astype(jnp.bfloat16)
    wu = w13[:, INTER:, :].astype(jnp.bfloat16)
    w2b = w2.astype(jnp.bfloat16)

    p, w, eot = pl.pallas_call(
        _route_kernel,
        in_specs=[
            pl.BlockSpec((T, HIDDEN), lambda: (0, 0)),
            pl.BlockSpec((NUM_EXPERTS, HIDDEN), lambda: (0, 0)),
        ],
        out_specs=[
            pl.BlockSpec((TOP_K, T), lambda: (0, 0)),
            pl.BlockSpec((TOP_K, T), lambda: (0, 0)),
            pl.BlockSpec((1, NT), lambda: (0, 0)),
        ],
        out_shape=[
            jax.ShapeDtypeStruct((TOP_K, T), jnp.int32),
            jax.ShapeDtypeStruct((TOP_K, T), jnp.float32),
            jax.ShapeDtypeStruct((1, NT), jnp.int32),
        ],
    )(x, router_w)

    out = pl.pallas_call(
        _gmm_kernel,
        grid_spec=pltpu.PrefetchScalarGridSpec(
            num_scalar_prefetch=1,
            grid=(NT,),
            in_specs=[
                pl.BlockSpec((TOP_K, T), lambda j, eot: (0, 0)),
                pl.BlockSpec((TOP_K, T), lambda j, eot: (0, 0)),
                pl.BlockSpec((T, HIDDEN), lambda j, eot: (0, 0)),
                pl.BlockSpec((1, INTER, HIDDEN), lambda j, eot: (eot[j], 0, 0)),
                pl.BlockSpec((1, INTER, HIDDEN), lambda j, eot: (eot[j], 0, 0)),
                pl.BlockSpec((1, HIDDEN, INTER), lambda j, eot: (eot[j], 0, 0)),
            ],
            out_specs=pl.BlockSpec((T, HIDDEN), lambda j, eot: (0, 0)),
        ),
        out_shape=jax.ShapeDtypeStruct((T, HIDDEN), jnp.float32),
    )(eot.reshape(NT), p, w, xb, wg, wu, w2b)
    return out.reshape(orig_shape)


# speedup vs baseline: 1.3392x; 1.3392x over previous
import jax
import jax.numpy as jnp
from jax.experimental import pallas as pl
from jax.experimental.pallas import tpu as pltpu

NUM_EXPERTS = 8
TOP_K = 2
HIDDEN = 768
INTER = 2048
T = 2048

TR = 256
NT = (2 * T + NUM_EXPERTS * TR) // TR
P = NT * TR


def _cumsum_manual(a, axis, n):
    idx = jax.lax.broadcasted_iota(jnp.int32, a.shape, axis)
    k = 1
    while k < n:
        a = a + jnp.where(idx >= k, jnp.roll(a, k, axis=axis), 0)
        k *= 2
    return a


def _route_kernel(x_ref, rw_ref, p_ref, w_ref, meta_ref):
    logits = jax.lax.dot_general(
        rw_ref[...], x_ref[...], (((1,), (1,)), ((), ())),
        preferred_element_type=jnp.float32)
    m = jnp.max(logits, axis=0, keepdims=True)
    ex = jnp.exp(logits - m)
    probs = ex / jnp.sum(ex, axis=0, keepdims=True)

    eidx = jax.lax.broadcasted_iota(jnp.int32, (NUM_EXPERTS, T), 0)
    w0 = jnp.max(probs, axis=0, keepdims=True)
    e0 = jnp.min(jnp.where(probs == w0, eidx, NUM_EXPERTS), axis=0,
                 keepdims=True)
    oh0 = (eidx == e0).astype(jnp.int32)
    probs2 = jnp.where(oh0 == 1, -1.0, probs)
    w1 = jnp.max(probs2, axis=0, keepdims=True)
    e1 = jnp.min(jnp.where(probs2 == w1, eidx, NUM_EXPERTS), axis=0,
                 keepdims=True)
    oh1 = (eidx == e1).astype(jnp.int32)

    cum0 = _cumsum_manual(oh0, 1, T)
    cum1 = _cumsum_manual(oh1, 1, T)
    counts0 = jnp.sum(oh0, axis=1, keepdims=True)
    counts = counts0 + jnp.sum(oh1, axis=1, keepdims=True)
    nt = (counts + TR - 1) // TR
    cnt_pad = nt * TR
    pad_off = _cumsum_manual(cnt_pad, 0, NUM_EXPERTS) - cnt_pad

    rank0 = jnp.sum(oh0 * (cum0 - 1), axis=0, keepdims=True)
    rank1 = jnp.sum(oh1 * (cum1 - 1 + counts0), axis=0, keepdims=True)
    base0 = jnp.sum(oh0 * pad_off, axis=0, keepdims=True)
    base1 = jnp.sum(oh1 * pad_off, axis=0, keepdims=True)

    p_ref[0:1, :] = base0 + rank0
    p_ref[1:2, :] = base1 + rank1
    w_ref[0:1, :] = w0
    w_ref[1:2, :] = w1
    meta_ref[0:NUM_EXPERTS, :] = pad_off
    meta_ref[NUM_EXPERTS:, :] = nt


def _gmm_kernel(meta_ref, p_ref, w_ref, xb_ref, wg_ref, wu_ref, w2_ref,
                out_ref):
    e = pl.program_id(0)

    @pl.when(e == 0)
    def _init():
        out_ref[...] = jnp.zeros_like(out_ref)

    base = meta_ref[e]
    ntiles = meta_ref[NUM_EXPERTS + e]
    riota = jax.lax.broadcasted_iota(jnp.int32, (TR, T), 0)
    p0 = p_ref[0:1, :]
    p1 = p_ref[1:2, :]
    wf0 = w_ref[0:1, :]
    wf1 = w_ref[1:2, :]

    def body(it, carry):
        rr = riota + (base + it * TR)
        m0 = p0 == rr
        m1 = p1 == rr
        ohu = (m0 | m1).astype(jnp.bfloat16)
        ohw = (jnp.where(m0, wf0, 0.0)
               + jnp.where(m1, wf1, 0.0)).astype(jnp.bfloat16)

        xs = jax.lax.dot_general(
            ohu, xb_ref[...], (((1,), (0,)), ((), ())),
            preferred_element_type=jnp.float32).astype(jnp.bfloat16)
        g = jax.lax.dot_general(
            xs, wg_ref[0], (((1,), (1,)), ((), ())),
            preferred_element_type=jnp.float32)
        u = jax.lax.dot_general(
            xs, wu_ref[0], (((1,), (1,)), ((), ())),
            preferred_element_type=jnp.float32)
        h = (g * jax.lax.logistic(g) * u).astype(jnp.bfloat16)
        y = jax.lax.dot_general(
            h, w2_ref[0], (((1,), (1,)), ((), ())),
            preferred_element_type=jnp.float32).astype(jnp.bfloat16)
        out_ref[...] += jax.lax.dot_general(
            ohw, y, (((0,), (0,)), ((), ())),
            preferred_element_type=jnp.float32)
        return carry

    jax.lax.fori_loop(0, ntiles, body, 0)


@jax.jit
def kernel(hidden_states, router_w, w13, w2):
    orig_shape = hidden_states.shape
    x = hidden_states.reshape(T, HIDDEN)
    xb = x.astype(jnp.bfloat16)
    wg = w13[:, :INTER, :].astype(jnp.bfloat16)
    wu = w13[:, INTER:, :].astype(jnp.bfloat16)
    w2b = w2.astype(jnp.bfloat16)

    p, w, meta = pl.pallas_call(
        _route_kernel,
        in_specs=[
            pl.BlockSpec((T, HIDDEN), lambda: (0, 0)),
            pl.BlockSpec((NUM_EXPERTS, HIDDEN), lambda: (0, 0)),
        ],
        out_specs=[
            pl.BlockSpec((TOP_K, T), lambda: (0, 0)),
            pl.BlockSpec((TOP_K, T), lambda: (0, 0)),
            pl.BlockSpec((2 * NUM_EXPERTS, 1), lambda: (0, 0)),
        ],
        out_shape=[
            jax.ShapeDtypeStruct((TOP_K, T), jnp.int32),
            jax.ShapeDtypeStruct((TOP_K, T), jnp.float32),
            jax.ShapeDtypeStruct((2 * NUM_EXPERTS, 1), jnp.int32),
        ],
    )(x, router_w)

    out = pl.pallas_call(
        _gmm_kernel,
        grid_spec=pltpu.PrefetchScalarGridSpec(
            num_scalar_prefetch=1,
            grid=(NUM_EXPERTS,),
            in_specs=[
                pl.BlockSpec((TOP_K, T), lambda e, meta: (0, 0)),
                pl.BlockSpec((TOP_K, T), lambda e, meta: (0, 0)),
                pl.BlockSpec((T, HIDDEN), lambda e, meta: (0, 0)),
                pl.BlockSpec((1, INTER, HIDDEN), lambda e, meta: (e, 0, 0)),
                pl.BlockSpec((1, INTER, HIDDEN), lambda e, meta: (e, 0, 0)),
                pl.BlockSpec((1, HIDDEN, INTER), lambda e, meta: (e, 0, 0)),
            ],
            out_specs=pl.BlockSpec((T, HIDDEN), lambda e, meta: (0, 0)),
        ),
        out_shape=jax.ShapeDtypeStruct((T, HIDDEN), jnp.float32),
    )(meta.reshape(2 * NUM_EXPERTS), p, w, xb, wg, wu, w2b)
    return out.reshape(orig_shape)


# speedup vs baseline: 1.6334x; 1.2197x over previous
import jax
import jax.numpy as jnp
from jax.experimental import pallas as pl
from jax.experimental.pallas import tpu as pltpu

NUM_EXPERTS = 8
TOP_K = 2
HIDDEN = 768
INTER = 2048
T = 2048

TR = 256
NT = (2 * T + NUM_EXPERTS * TR) // TR
P = NT * TR


def _cumsum_manual(a, axis, n):
    idx = jax.lax.broadcasted_iota(jnp.int32, a.shape, axis)
    k = 1
    while k < n:
        a = a + jnp.where(idx >= k, jnp.roll(a, k, axis=axis), 0)
        k *= 2
    return a


def _route_kernel(x_ref, rw_ref, p_ref, w_ref, meta_ref):
    logits = jax.lax.dot_general(
        rw_ref[...], x_ref[...], (((1,), (1,)), ((), ())),
        preferred_element_type=jnp.float32)
    m = jnp.max(logits, axis=0, keepdims=True)
    ex = jnp.exp(logits - m)
    probs = ex / jnp.sum(ex, axis=0, keepdims=True)

    eidx = jax.lax.broadcasted_iota(jnp.int32, (NUM_EXPERTS, T), 0)
    w0 = jnp.max(probs, axis=0, keepdims=True)
    e0 = jnp.min(jnp.where(probs == w0, eidx, NUM_EXPERTS), axis=0,
                 keepdims=True)
    oh0 = (eidx == e0).astype(jnp.int32)
    probs2 = jnp.where(oh0 == 1, -1.0, probs)
    w1 = jnp.max(probs2, axis=0, keepdims=True)
    e1 = jnp.min(jnp.where(probs2 == w1, eidx, NUM_EXPERTS), axis=0,
                 keepdims=True)
    oh1 = (eidx == e1).astype(jnp.int32)

    cum0 = _cumsum_manual(oh0, 1, T)
    cum1 = _cumsum_manual(oh1, 1, T)
    counts0 = jnp.sum(oh0, axis=1, keepdims=True)
    counts = counts0 + jnp.sum(oh1, axis=1, keepdims=True)
    nt = (counts + TR - 1) // TR
    cnt_pad = nt * TR
    pad_off = _cumsum_manual(cnt_pad, 0, NUM_EXPERTS) - cnt_pad

    rank0 = jnp.sum(oh0 * (cum0 - 1), axis=0, keepdims=True)
    rank1 = jnp.sum(oh1 * (cum1 - 1 + counts0), axis=0, keepdims=True)
    base0 = jnp.sum(oh0 * pad_off, axis=0, keepdims=True)
    base1 = jnp.sum(oh1 * pad_off, axis=0, keepdims=True)

    p_ref[0:1, :] = base0 + rank0
    p_ref[1:2, :] = base1 + rank1
    w_ref[0:1, :] = w0
    w_ref[1:2, :] = w1
    meta_ref[0:NUM_EXPERTS, :] = pad_off
    meta_ref[NUM_EXPERTS:, :] = nt


def _gmm_kernel(meta_ref, p_ref, w_ref, xb_ref, wg_ref, wu_ref, w2_ref,
                out_ref):
    e = pl.program_id(0)

    @pl.when(e == 0)
    def _init():
        out_ref[...] = jnp.zeros_like(out_ref)

    base = meta_ref[e]
    ntiles = meta_ref[NUM_EXPERTS + e]
    riota = jax.lax.broadcasted_iota(jnp.int32, (TR, T), 0)
    p0 = p_ref[0:1, :]
    p1 = p_ref[1:2, :]
    wf0 = w_ref[0:1, :]
    wf1 = w_ref[1:2, :]

    def body(it, carry):
        rr = riota + (base + it * TR)
        m0 = p0 == rr
        m1 = p1 == rr
        ohu = (m0 | m1).astype(jnp.float32)
        ohw = (jnp.where(m0, wf0, 0.0)
               + jnp.where(m1, wf1, 0.0))

        xs = jax.lax.dot_general(
            ohu, xb_ref[...], (((1,), (0,)), ((), ())),
            preferred_element_type=jnp.float32)
        g = jax.lax.dot_general(
            xs, wg_ref[0], (((1,), (1,)), ((), ())),
            preferred_element_type=jnp.float32)
        u = jax.lax.dot_general(
            xs, wu_ref[0], (((1,), (1,)), ((), ())),
            preferred_element_type=jnp.float32)
        h = g * jax.lax.logistic(g) * u
        y = jax.lax.dot_general(
            h, w2_ref[0], (((1,), (1,)), ((), ())),
            preferred_element_type=jnp.float32)
        out_ref[...] += jax.lax.dot_general(
            ohw, y, (((0,), (0,)), ((), ())),
            preferred_element_type=jnp.float32)
        return carry

    jax.lax.fori_loop(0, ntiles, body, 0)


@jax.jit
def kernel(hidden_states, router_w, w13, w2):
    orig_shape = hidden_states.shape
    x = hidden_states.reshape(T, HIDDEN)
    wg = w13[:, :INTER, :]
    wu = w13[:, INTER:, :]
    w2b = w2
    xb = x

    p, w, meta = pl.pallas_call(
        _route_kernel,
        in_specs=[
            pl.BlockSpec((T, HIDDEN), lambda: (0, 0)),
            pl.BlockSpec((NUM_EXPERTS, HIDDEN), lambda: (0, 0)),
        ],
        out_specs=[
            pl.BlockSpec((TOP_K, T), lambda: (0, 0)),
            pl.BlockSpec((TOP_K, T), lambda: (0, 0)),
            pl.BlockSpec((2 * NUM_EXPERTS, 1), lambda: (0, 0)),
        ],
        out_shape=[
            jax.ShapeDtypeStruct((TOP_K, T), jnp.int32),
            jax.ShapeDtypeStruct((TOP_K, T), jnp.float32),
            jax.ShapeDtypeStruct((2 * NUM_EXPERTS, 1), jnp.int32),
        ],
    )(x, router_w)

    out = pl.pallas_call(
        _gmm_kernel,
        grid_spec=pltpu.PrefetchScalarGridSpec(
            num_scalar_prefetch=1,
            grid=(NUM_EXPERTS,),
            in_specs=[
                pl.BlockSpec((TOP_K, T), lambda e, meta: (0, 0)),
                pl.BlockSpec((TOP_K, T), lambda e, meta: (0, 0)),
                pl.BlockSpec((T, HIDDEN), lambda e, meta: (0, 0)),
                pl.BlockSpec((1, INTER, HIDDEN), lambda e, meta: (e, 0, 0)),
                pl.BlockSpec((1, INTER, HIDDEN), lambda e, meta: (e, 0, 0)),
                pl.BlockSpec((1, HIDDEN, INTER), lambda e, meta: (e, 0, 0)),
            ],
            out_specs=pl.BlockSpec((T, HIDDEN), lambda e, meta: (0, 0)),
        ),
        out_shape=jax.ShapeDtypeStruct((T, HIDDEN), jnp.float32),
    )(meta.reshape(2 * NUM_EXPERTS), p, w, xb, wg, wu, w2b)
    return out.reshape(orig_shape)
